# baseline (speedup 1.0000x reference)
import functools

import jax
import jax.numpy as jnp
from jax import lax
from jax.experimental import pallas as pl
from jax.experimental.pallas import tpu as pltpu
from jax.experimental.pallas import tpu_sc as plsc

B = 16
MAX_LEN = 4096
TOTAL = B * (MAX_LEN // 2)
FLAT = TOTAL * 4
ROW_W = MAX_LEN * 4
HALF_W = ROW_W // 2
WIN = 17 * 512
W0_MAX = FLAT - WIN
NG = 16


def _body(xin_hbm, cu_hbm, out_hbm, cu_v, in_v, out_v, sem):
    cid = lax.axis_index("c")
    sid = lax.axis_index("s")
    b = sid
    g0 = cid * NG

    pltpu.sync_copy(cu_hbm, cu_v)

    s = cu_v[pl.ds(b, 16)][0]
    e = cu_v[pl.ds(b + 1, 16)][0]
    length = e - s
    m = lax.rem(s, 128)
    sg = lax.div(s, 128)

    w0 = jnp.minimum((sg + g0) * 512, W0_MAX)
    delta = (sg + g0) - lax.div(w0, 512)

    pltpu.sync_copy(xin_hbm.at[pl.ds(w0, WIN)], in_v.at[pl.ds(0, WIN)])

    lanes = lax.iota(jnp.int32, 16)
    from_a = [(v * 16 + lanes) < (128 - m) for v in range(8)]

    def gblock(g):
        base_p = (delta + g) * 512 + m
        jg = (g0 + g) * 128
        valid = [(jg + v * 16 + lanes) < length for v in range(8)]
        for c in range(4):
            for v in range(8):
                off = base_p + c * 128 + v * 16
                p1 = jnp.minimum(off, WIN)
                p2 = jnp.minimum(off + 384, WIN)
                x1 = in_v[pl.ds(p1, 16)]
                x2 = in_v[pl.ds(p2, 16)]
                x = jnp.where(from_a[v], x1, x2)
                x = jnp.where(valid[v], x, -1.0)
                out_v[pl.ds(g * 512 + c * 128 + v * 16, 16)] = x

    out_base = b * ROW_W + g0 * 512
    plsc.parallel_loop(0, NG // 2)(gblock)
    cp1 = pltpu.async_copy(
        out_v.at[pl.ds(0, HALF_W // 2)],
        out_hbm.at[pl.ds(out_base, HALF_W // 2)], sem)
    plsc.parallel_loop(NG // 2, NG)(gblock)
    cp2 = pltpu.async_copy(
        out_v.at[pl.ds(HALF_W // 2, HALF_W // 2)],
        out_hbm.at[pl.ds(out_base + HALF_W // 2, HALF_W // 2)], sem)
    cp1.wait()
    cp2.wait()


@jax.jit
def _bbox_to_dense(xin, cu):
    mesh = plsc.VectorSubcoreMesh(core_axis_name="c", subcore_axis_name="s")
    run = functools.partial(
        pl.kernel,
        out_type=jax.ShapeDtypeStruct((B * ROW_W,), jnp.float32),
        mesh=mesh,
        scratch_types=[
            pltpu.VMEM((B + 1,), jnp.int32),
            pltpu.VMEM((WIN + 16,), jnp.float32),
            pltpu.VMEM((HALF_W,), jnp.float32),
            pltpu.SemaphoreType.DMA,
        ],
    )(_body)
    return run(xin, cu)


def kernel(bbox_values, cu_seqlens, keep_ragged):
    xin = bbox_values.reshape(256, 128, 4).transpose(0, 2, 1).reshape(-1)
    out = _bbox_to_dense(xin, cu_seqlens.astype(jnp.int32))
    return out.reshape(B, 32, 4, 128).transpose(0, 1, 3, 2).reshape(B, MAX_LEN, 4)


# speedup vs baseline: 1.0588x; 1.0588x over previous
import functools

import jax
import jax.numpy as jnp
from jax import lax
from jax.experimental import pallas as pl
from jax.experimental.pallas import tpu as pltpu
from jax.experimental.pallas import tpu_sc as plsc

B = 16
MAX_LEN = 4096
TOTAL = B * (MAX_LEN // 2)
FLAT = TOTAL * 4
ROW_W = MAX_LEN * 4
HALF_W = ROW_W // 2
WIN = 17 * 512
W0_MAX = FLAT - WIN
NG = 16


def _body(xin_hbm, cu_hbm, out_hbm, cu_v, in_v, out_v, sem):
    cid = lax.axis_index("c")
    sid = lax.axis_index("s")
    b = sid
    g0 = cid * NG

    pltpu.sync_copy(cu_hbm, cu_v)

    s = cu_v[pl.ds(b, 16)][0]
    e = cu_v[pl.ds(b + 1, 16)][0]
    length = e - s
    m = lax.rem(s, 128)
    sg = lax.div(s, 128)

    w0 = jnp.minimum((sg + g0) * 512, W0_MAX)
    delta = (sg + g0) - lax.div(w0, 512)

    pltpu.sync_copy(xin_hbm.at[pl.ds(w0, WIN)], in_v.at[pl.ds(0, WIN)])

    lanes = lax.iota(jnp.int32, 16)
    from_a = [(v * 16 + lanes) < (128 - m) for v in range(8)]

    def gblock(g, carry=None):
        base_p = (delta + g) * 512 + m
        jg = (g0 + g) * 128
        valid = [(jg + v * 16 + lanes) < length for v in range(8)]
        for c in range(4):
            for v in range(8):
                off = base_p + c * 128 + v * 16
                p1 = jnp.minimum(off, WIN)
                p2 = jnp.minimum(off + 384, WIN)
                x1 = in_v[pl.ds(p1, 16)]
                x2 = in_v[pl.ds(p2, 16)]
                x = jnp.where(from_a[v], x1, x2)
                x = jnp.where(valid[v], x, -1.0)
                out_v[pl.ds(g * 512 + c * 128 + v * 16, 16)] = x

    out_base = b * ROW_W + g0 * 512
    lax.fori_loop(0, NG // 2, gblock, None)
    cp1 = pltpu.async_copy(
        out_v.at[pl.ds(0, HALF_W // 2)],
        out_hbm.at[pl.ds(out_base, HALF_W // 2)], sem)
    lax.fori_loop(NG // 2, NG, gblock, None)
    cp2 = pltpu.async_copy(
        out_v.at[pl.ds(HALF_W // 2, HALF_W // 2)],
        out_hbm.at[pl.ds(out_base + HALF_W // 2, HALF_W // 2)], sem)
    cp1.wait()
    cp2.wait()


@jax.jit
def _bbox_to_dense(xin, cu):
    mesh = plsc.VectorSubcoreMesh(core_axis_name="c", subcore_axis_name="s")
    run = functools.partial(
        pl.kernel,
        out_type=jax.ShapeDtypeStruct((B * ROW_W,), jnp.float32),
        mesh=mesh,
        scratch_types=[
            pltpu.VMEM((B + 1,), jnp.int32),
            pltpu.VMEM((WIN + 16,), jnp.float32),
            pltpu.VMEM((HALF_W,), jnp.float32),
            pltpu.SemaphoreType.DMA,
        ],
    )(_body)
    return run(xin, cu)


def kernel(bbox_values, cu_seqlens, keep_ragged):
    xin = bbox_values.reshape(256, 128, 4).transpose(0, 2, 1).reshape(-1)
    out = _bbox_to_dense(xin, cu_seqlens.astype(jnp.int32))
    return out.reshape(B, 32, 4, 128).transpose(0, 1, 3, 2).reshape(B, MAX_LEN, 4)


# speedup vs baseline: 1.1088x; 1.0472x over previous
import functools

import jax
import jax.numpy as jnp
from jax import lax
from jax.experimental import pallas as pl
from jax.experimental.pallas import tpu as pltpu
from jax.experimental.pallas import tpu_sc as plsc

B = 16
MAX_LEN = 4096
TOTAL = B * (MAX_LEN // 2)
FLAT = TOTAL * 4
ROW_W = MAX_LEN * 4
HALF_W = ROW_W // 2
WIN = 17 * 512
W0_MAX = FLAT - WIN
NG = 16


def _body(xin_hbm, cu_hbm, out_hbm, cu_v, in_v, out_v, sem):
    cid = lax.axis_index("c")
    sid = lax.axis_index("s")
    b = sid
    g0 = cid * NG

    pltpu.sync_copy(cu_hbm, cu_v)

    s = cu_v[pl.ds(b, 16)][0]
    e = cu_v[pl.ds(b + 1, 16)][0]
    length = e - s
    m = lax.rem(s, 128)
    sg = lax.div(s, 128)

    w0 = jnp.minimum((sg + g0) * 512, W0_MAX)
    delta = (sg + g0) - lax.div(w0, 512)

    pltpu.sync_copy(xin_hbm.at[pl.ds(w0, WIN)], in_v.at[pl.ds(0, WIN)])

    lanes = lax.iota(jnp.int32, 16)
    from_a = [(v * 16 + lanes) < (128 - m) for v in range(8)]

    def gblock(gc, carry=None):
        g = gc // 4
        c = gc % 4
        base_p = (delta + g) * 512 + m
        jg = (g0 + g) * 128
        for v in range(8):
            off = base_p + c * 128 + v * 16
            p1 = jnp.minimum(off, WIN)
            p2 = jnp.minimum(off + 384, WIN)
            x1 = in_v[pl.ds(p1, 16)]
            x2 = in_v[pl.ds(p2, 16)]
            x = jnp.where(from_a[v], x1, x2)
            x = jnp.where((jg + v * 16 + lanes) < length, x, -1.0)
            out_v[pl.ds(g * 512 + c * 128 + v * 16, 16)] = x

    out_base = b * ROW_W + g0 * 512
    lax.fori_loop(0, NG * 4, gblock, None)
    pltpu.sync_copy(out_v, out_hbm.at[pl.ds(out_base, HALF_W)])


@jax.jit
def _bbox_to_dense(xin, cu):
    mesh = plsc.VectorSubcoreMesh(core_axis_name="c", subcore_axis_name="s")
    run = functools.partial(
        pl.kernel,
        out_type=jax.ShapeDtypeStruct((B * ROW_W,), jnp.float32),
        mesh=mesh,
        scratch_types=[
            pltpu.VMEM((B + 1,), jnp.int32),
            pltpu.VMEM((WIN + 16,), jnp.float32),
            pltpu.VMEM((HALF_W,), jnp.float32),
            pltpu.SemaphoreType.DMA,
        ],
    )(_body)
    return run(xin, cu)


def kernel(bbox_values, cu_seqlens, keep_ragged):
    xin = bbox_values.reshape(256, 128, 4).transpose(0, 2, 1).reshape(-1)
    out = _bbox_to_dense(xin, cu_seqlens.astype(jnp.int32))
    return out.reshape(B, 32, 4, 128).transpose(0, 1, 3, 2).reshape(B, MAX_LEN, 4)


# speedup vs baseline: 1.1194x; 1.0096x over previous
import functools

import jax
import jax.numpy as jnp
from jax import lax
from jax.experimental import pallas as pl
from jax.experimental.pallas import tpu as pltpu
from jax.experimental.pallas import tpu_sc as plsc

B = 16
MAX_LEN = 4096
TOTAL = B * (MAX_LEN // 2)
FLAT = TOTAL * 4
ROW_W = MAX_LEN * 4
HALF_W = ROW_W // 2
WIN = 17 * 512
W0_MAX = FLAT - WIN
NG = 16


def _body(xin_hbm, cu_hbm, out_hbm, cu_v, in_v, out_v, sem):
    cid = lax.axis_index("c")
    sid = lax.axis_index("s")
    b = sid
    g0 = cid * NG

    pltpu.sync_copy(cu_hbm, cu_v)

    s = cu_v[pl.ds(b, 16)][0]
    e = cu_v[pl.ds(b + 1, 16)][0]
    length = e - s
    m = lax.rem(s, 128)
    sg = lax.div(s, 128)

    w0 = jnp.minimum((sg + g0) * 512, W0_MAX)
    delta = (sg + g0) - lax.div(w0, 512)

    pltpu.sync_copy(xin_hbm.at[pl.ds(w0, WIN)], in_v.at[pl.ds(0, WIN)])

    lanes = lax.iota(jnp.int32, 16)
    from_a = [(v * 16 + lanes) < (128 - m) for v in range(8)]

    def gblock(i, carry=None):
        g = i // 8
        cv = i % 8
        c = cv // 2
        base_p = (delta + g) * 512 + m
        jg = (g0 + g) * 128
        for vh in range(4):
            v = (cv % 2) * 4 + vh
            kpos = v * 16 + lanes
            off = base_p + c * 128 + v * 16
            p1 = jnp.minimum(off, WIN)
            p2 = jnp.minimum(off + 384, WIN)
            x1 = in_v[pl.ds(p1, 16)]
            x2 = in_v[pl.ds(p2, 16)]
            x = jnp.where(kpos < 128 - m, x1, x2)
            x = jnp.where(jg + kpos < length, x, -1.0)
            out_v[pl.ds(g * 512 + c * 128 + v * 16, 16)] = x

    out_base = b * ROW_W + g0 * 512
    lax.fori_loop(0, NG * 8, gblock, None)
    pltpu.sync_copy(out_v, out_hbm.at[pl.ds(out_base, HALF_W)])


@jax.jit
def _bbox_to_dense(xin, cu):
    mesh = plsc.VectorSubcoreMesh(core_axis_name="c", subcore_axis_name="s")
    run = functools.partial(
        pl.kernel,
        out_type=jax.ShapeDtypeStruct((B * ROW_W,), jnp.float32),
        mesh=mesh,
        scratch_types=[
            pltpu.VMEM((B + 1,), jnp.int32),
            pltpu.VMEM((WIN + 16,), jnp.float32),
            pltpu.VMEM((HALF_W,), jnp.float32),
            pltpu.SemaphoreType.DMA,
        ],
    )(_body)
    return run(xin, cu)


def kernel(bbox_values, cu_seqlens, keep_ragged):
    xin = bbox_values.reshape(256, 128, 4).transpose(0, 2, 1).reshape(-1)
    out = _bbox_to_dense(xin, cu_seqlens.astype(jnp.int32))
    return out.reshape(B, 32, 4, 128).transpose(0, 1, 3, 2).reshape(B, MAX_LEN, 4)


# speedup vs baseline: 1.1282x; 1.0079x over previous
import functools

import jax
import jax.numpy as jnp
from jax import lax
from jax.experimental import pallas as pl
from jax.experimental.pallas import tpu as pltpu
from jax.experimental.pallas import tpu_sc as plsc

B = 16
MAX_LEN = 4096
TOTAL = B * (MAX_LEN // 2)
FLAT = TOTAL * 4
ROW_W = MAX_LEN * 4
HALF_W = ROW_W // 2
WIN = 17 * 512
W0_MAX = FLAT - WIN
NG = 16


def _body(xin_hbm, cu_hbm, out_hbm, cu_v, in_v, out_v, sem):
    cid = lax.axis_index("c")
    sid = lax.axis_index("s")
    b = sid
    g0 = cid * NG

    pltpu.sync_copy(cu_hbm, cu_v)

    s = cu_v[pl.ds(b, 16)][0]
    e = cu_v[pl.ds(b + 1, 16)][0]
    length = e - s
    m = lax.rem(s, 128)
    sg = lax.div(s, 128)

    w0 = jnp.minimum((sg + g0) * 512, W0_MAX)
    delta = (sg + g0) - lax.div(w0, 512)

    pltpu.sync_copy(xin_hbm.at[pl.ds(w0, WIN)], in_v.at[pl.ds(0, WIN)])

    lanes = lax.iota(jnp.int32, 16)
    from_a = [(v * 16 + lanes) < (128 - m) for v in range(8)]

    def gblock(i, carry=None):
        g = i // 32
        u = i % 32
        c = u // 8
        v = u % 8
        base_p = (delta + g) * 512 + m
        jg = (g0 + g) * 128
        kpos = v * 16 + lanes
        off = base_p + c * 128 + v * 16
        p1 = jnp.minimum(off, WIN)
        p2 = jnp.minimum(off + 384, WIN)
        x1 = in_v[pl.ds(p1, 16)]
        x2 = in_v[pl.ds(p2, 16)]
        x = jnp.where(kpos < 128 - m, x1, x2)
        x = jnp.where(jg + kpos < length, x, -1.0)
        out_v[pl.ds(g * 512 + c * 128 + v * 16, 16)] = x

    out_base = b * ROW_W + g0 * 512
    lax.fori_loop(0, NG * 32, gblock, None)
    pltpu.sync_copy(out_v, out_hbm.at[pl.ds(out_base, HALF_W)])


@jax.jit
def _bbox_to_dense(xin, cu):
    mesh = plsc.VectorSubcoreMesh(core_axis_name="c", subcore_axis_name="s")
    run = functools.partial(
        pl.kernel,
        out_type=jax.ShapeDtypeStruct((B * ROW_W,), jnp.float32),
        mesh=mesh,
        scratch_types=[
            pltpu.VMEM((B + 1,), jnp.int32),
            pltpu.VMEM((WIN + 16,), jnp.float32),
            pltpu.VMEM((HALF_W,), jnp.float32),
            pltpu.SemaphoreType.DMA,
        ],
    )(_body)
    return run(xin, cu)


def kernel(bbox_values, cu_seqlens, keep_ragged):
    xin = bbox_values.reshape(256, 128, 4).transpose(0, 2, 1).reshape(-1)
    out = _bbox_to_dense(xin, cu_seqlens.astype(jnp.int32))
    return out.reshape(B, 32, 4, 128).transpose(0, 1, 3, 2).reshape(B, MAX_LEN, 4)
